# Initial kernel scaffold; baseline (speedup 1.0000x reference)
import jax
import jax.numpy as jnp
from jax.experimental import pallas as pl


def kernel(src0, src1, src2, src3, L0_sa_in_w, L0_sa_in_b, L0_sa_out_w, L0_sa_out_b, L0_ca_in_w, L0_ca_in_b, L0_ca_out_w, L0_ca_out_b, L0_lin1_w, L0_lin1_b, L0_lin2_w, L0_lin2_b, L0_n1_w, L0_n1_b, L0_n2_w, L0_n2_b, L0_n3_w, L0_n3_b, L1_sa_in_w, L1_sa_in_b, L1_sa_out_w, L1_sa_out_b, L1_ca_in_w, L1_ca_in_b, L1_ca_out_w, L1_ca_out_b, L1_lin1_w, L1_lin1_b, L1_lin2_w, L1_lin2_b, L1_n1_w, L1_n1_b, L1_n2_w, L1_n2_b, L1_n3_w, L1_n3_b, L2_sa_in_w, L2_sa_in_b, L2_sa_out_w, L2_sa_out_b, L2_ca_in_w, L2_ca_in_b, L2_ca_out_w, L2_ca_out_b, L2_lin1_w, L2_lin1_b, L2_lin2_w, L2_lin2_b, L2_n1_w, L2_n1_b, L2_n2_w, L2_n2_b, L2_n3_w, L2_n3_b):
    raise NotImplementedError("write your pallas kernel here")


# speedup vs baseline: 17.4555x; 17.4555x over previous
import functools
import math

import jax
import jax.numpy as jnp
from jax import lax
from jax.experimental import pallas as pl
from jax.experimental.pallas import tpu as pltpu
from jax.experimental.pallas import tpu_sc as plsc

_D = 256
_NH = 8
_DH = 32
_K = 4
_DFF = 1024
_BS = 4
_HWS = (64, 256, 1024)
_NMEM = 5440
_NQB = sum(_HWS)
_NQ = _BS * _NQB
_BQ_SIM = 192
_SIM_BPB = _NQB // _BQ_SIM
_BQ_DEC = 256
_NROWS = _NQ * _K
_NW = 32
_ROWS_PER_W = _NROWS // _NW
_CH = 168
_NCH = _ROWS_PER_W // _CH

_HIGH = lax.Precision.HIGHEST


def _mm(x, w):
    return lax.dot_general(x, w, (((1,), (1,)), ((), ())),
                           preferred_element_type=jnp.float32, precision=_HIGH)


def _sim_topk_kernel(q_ref, mem_ref, idx_ref):
    q = q_ref[...]
    mem = mem_ref[0]
    s = _mm(q, mem)
    col = lax.broadcasted_iota(jnp.int32, s.shape, 1)
    picks = []
    for _ in range(_K):
        m = jnp.max(s, axis=1, keepdims=True)
        hit = s >= m
        idx_t = jnp.min(jnp.where(hit, col, jnp.int32(2**30)), axis=1)
        picks.append(idx_t[:, None])
        s = jnp.where(col == idx_t[:, None], jnp.float32(-3e38), s)
    idx_ref[...] = jnp.concatenate(picks, axis=1)


def _sim_topk(q_bm, all_feas):
    return pl.pallas_call(
        _sim_topk_kernel,
        grid=(_NQ // _BQ_SIM,),
        in_specs=[
            pl.BlockSpec((_BQ_SIM, _D), lambda i: (i, 0)),
            pl.BlockSpec((1, _NMEM, _D), lambda i: (i // _SIM_BPB, 0, 0)),
        ],
        out_specs=pl.BlockSpec((_BQ_SIM, _K), lambda i: (i, 0)),
        out_shape=jax.ShapeDtypeStruct((_NQ, _K), jnp.int32),
    )(q_bm, all_feas)


def _sc_gather(table, idx_flat):
    mesh = plsc.VectorSubcoreMesh(core_axis_name="c", subcore_axis_name="s")

    @functools.partial(
        pl.kernel,
        mesh=mesh,
        out_type=jax.ShapeDtypeStruct((_NROWS, _D), jnp.float32),
        scratch_types=[
            pltpu.VMEM((_CH,), jnp.int32),
            pltpu.VMEM((_CH, _D), jnp.float32),
            pltpu.SemaphoreType.DMA,
        ],
    )
    def gather_k(table_hbm, idx_hbm, out_hbm, idx_v, rows_v, sem):
        wid = lax.axis_index("s") * 2 + lax.axis_index("c")
        base = wid * _ROWS_PER_W
        for c in range(_NCH):
            off = base + c * _CH
            pltpu.sync_copy(idx_hbm.at[pl.ds(off, _CH)], idx_v)
            pltpu.async_copy(table_hbm.at[idx_v], rows_v, sem).wait()
            pltpu.sync_copy(rows_v, out_hbm.at[pl.ds(off, _CH)])

    return gather_k(table, idx_flat)


def _ln(x, w, b):
    m = jnp.mean(x, axis=1, keepdims=True)
    d = x - m
    v = jnp.mean(d * d, axis=1, keepdims=True)
    return d / jnp.sqrt(v + 1e-5) * w + b


def _dec_kernel(x_ref, kv_ref, sa_wv, sa_ow, ca_wq, ca_wk, ca_wv, ca_ow,
                l1w, l1b, l2w, vecs, o_ref):
    x = x_ref[...]
    V = vecs[0]
    bv_sa = V[0:1]; ob_sa = V[1:2]; n1w = V[2:3]; n1b = V[3:4]
    bq = V[4:5]; bk = V[5:6]; bv = V[6:7]; ob_ca = V[7:8]
    n2w = V[8:9]; n2b = V[9:10]; b_l2 = V[10:11]; n3w = V[11:12]; n3b = V[12:13]

    sa = _mm(_mm(x, sa_wv[0]) + bv_sa, sa_ow[0]) + ob_sa
    t1 = _ln(x + sa, n1w, n1b)

    q = _mm(t1, ca_wq[0]) + bq
    hrow = lax.broadcasted_iota(jnp.int32, (_D, _NH), 0) // _DH
    hcol = lax.broadcasted_iota(jnp.int32, (_D, _NH), 1)
    H = (hrow == hcol).astype(jnp.float32)
    scale = 1.0 / math.sqrt(_DH)
    ss, vs = [], []
    for j in range(_K):
        g = kv_ref[j]
        kp = _mm(g, ca_wk[0]) + bk
        vs.append(_mm(g, ca_wv[0]) + bv)
        sj = lax.dot_general(q * kp, H, (((1,), (0,)), ((), ())),
                             preferred_element_type=jnp.float32,
                             precision=_HIGH) * scale
        ss.append(sj)
    m = jnp.maximum(jnp.maximum(ss[0], ss[1]), jnp.maximum(ss[2], ss[3]))
    es = [jnp.exp(sj - m) for sj in ss]
    den = es[0] + es[1] + es[2] + es[3]
    att = jnp.zeros_like(x)
    for j in range(_K):
        wfull = lax.dot_general(es[j] / den, H, (((1,), (1,)), ((), ())),
                                preferred_element_type=jnp.float32,
                                precision=_HIGH)
        att = att + wfull * vs[j]
    ca = _mm(att, ca_ow[0]) + ob_ca
    t2 = _ln(t1 + ca, n2w, n2b)

    h = jnp.maximum(_mm(t2, l1w[0]) + l1b[0], 0.0)
    f = _mm(h, l2w[0]) + b_l2
    t3 = _ln(t2 + f, n3w, n3b)
    o_ref[...] = t3 + x


def _lvl_of(i):
    return jnp.int32(i >= 1) + jnp.int32(i >= 5)


def _decoder(q_lm, kv, ws):
    def wspec(a, b):
        return pl.BlockSpec((1, a, b), lambda i: (_lvl_of(i), 0, 0))
    return pl.pallas_call(
        _dec_kernel,
        grid=(_NQ // _BQ_DEC,),
        in_specs=[
            pl.BlockSpec((_BQ_DEC, _D), lambda i: (i, 0)),
            pl.BlockSpec((_K, _BQ_DEC, _D), lambda i: (0, i, 0)),
            wspec(_D, _D), wspec(_D, _D), wspec(_D, _D), wspec(_D, _D),
            wspec(_D, _D), wspec(_D, _D),
            wspec(_DFF, _D), wspec(1, _DFF), wspec(_D, _DFF),
            wspec(16, _D),
        ],
        out_specs=pl.BlockSpec((_BQ_DEC, _D), lambda i: (i, 0)),
        out_shape=jax.ShapeDtypeStruct((_NQ, _D), jnp.float32),
    )(q_lm, kv, *ws)


def kernel(src0, src1, src2, src3,
           L0_sa_in_w, L0_sa_in_b, L0_sa_out_w, L0_sa_out_b,
           L0_ca_in_w, L0_ca_in_b, L0_ca_out_w, L0_ca_out_b,
           L0_lin1_w, L0_lin1_b, L0_lin2_w, L0_lin2_b,
           L0_n1_w, L0_n1_b, L0_n2_w, L0_n2_b, L0_n3_w, L0_n3_b,
           L1_sa_in_w, L1_sa_in_b, L1_sa_out_w, L1_sa_out_b,
           L1_ca_in_w, L1_ca_in_b, L1_ca_out_w, L1_ca_out_b,
           L1_lin1_w, L1_lin1_b, L1_lin2_w, L1_lin2_b,
           L1_n1_w, L1_n1_b, L1_n2_w, L1_n2_b, L1_n3_w, L1_n3_b,
           L2_sa_in_w, L2_sa_in_b, L2_sa_out_w, L2_sa_out_b,
           L2_ca_in_w, L2_ca_in_b, L2_ca_out_w, L2_ca_out_b,
           L2_lin1_w, L2_lin1_b, L2_lin2_w, L2_lin2_b,
           L2_n1_w, L2_n1_b, L2_n2_w, L2_n2_b, L2_n3_w, L2_n3_b):
    srcs = [src0, src1, src2, src3]
    flat = [s.reshape(_BS, _D, -1).transpose(0, 2, 1) for s in srcs]
    all_feas = jnp.concatenate(flat, axis=1)
    q_lvl = [flat[3], flat[2], flat[1]]

    q_bm = jnp.concatenate(
        [jnp.concatenate([q_lvl[0][b], q_lvl[1][b], q_lvl[2][b]], 0)
         for b in range(_BS)], 0)
    idx_bm = _sim_topk(q_bm, all_feas)

    per_b = jnp.split(idx_bm, _BS, axis=0)
    offs = (0, 64, 320, 1344)
    idx_lm = jnp.concatenate(
        [jnp.concatenate([pb[offs[l]:offs[l + 1]] for pb in per_b], 0)
         for l in range(3)], 0)
    idx_flat = idx_lm.T.reshape(-1)

    kv = _sc_gather(all_feas[0], idx_flat).reshape(_K, _NQ, _D)

    q_lm = jnp.concatenate([q_lvl[l].reshape(-1, _D) for l in range(3)], 0)

    def stack3(a, b, c):
        return jnp.stack([a, b, c])

    sa_in = (L0_sa_in_w, L1_sa_in_w, L2_sa_in_w)
    ca_in = (L0_ca_in_w, L1_ca_in_w, L2_ca_in_w)
    ws = [
        stack3(*[w[2 * _D:] for w in sa_in]),
        stack3(L0_sa_out_w, L1_sa_out_w, L2_sa_out_w),
        stack3(*[w[:_D] for w in ca_in]),
        stack3(*[w[_D:2 * _D] for w in ca_in]),
        stack3(*[w[2 * _D:] for w in ca_in]),
        stack3(L0_ca_out_w, L1_ca_out_w, L2_ca_out_w),
        stack3(L0_lin1_w, L1_lin1_w, L2_lin1_w),
        stack3(L0_lin1_b, L1_lin1_b, L2_lin1_b).reshape(3, 1, _DFF),
        stack3(L0_lin2_w, L1_lin2_w, L2_lin2_w),
    ]
    zeros = jnp.zeros((_D,), jnp.float32)
    vec_rows = []
    for sa_b, sa_ob, ca_b, ca_ob, l2b, n1w, n1b, n2w, n2b, n3w, n3b in (
        (L0_sa_in_b, L0_sa_out_b, L0_ca_in_b, L0_ca_out_b, L0_lin2_b,
         L0_n1_w, L0_n1_b, L0_n2_w, L0_n2_b, L0_n3_w, L0_n3_b),
        (L1_sa_in_b, L1_sa_out_b, L1_ca_in_b, L1_ca_out_b, L1_lin2_b,
         L1_n1_w, L1_n1_b, L1_n2_w, L1_n2_b, L1_n3_w, L1_n3_b),
        (L2_sa_in_b, L2_sa_out_b, L2_ca_in_b, L2_ca_out_b, L2_lin2_b,
         L2_n1_w, L2_n1_b, L2_n2_w, L2_n2_b, L2_n3_w, L2_n3_b),
    ):
        vec_rows.append(jnp.stack([
            sa_b[2 * _D:], sa_ob, n1w, n1b,
            ca_b[:_D], ca_b[_D:2 * _D], ca_b[2 * _D:], ca_ob,
            n2w, n2b, l2b, n3w, n3b, zeros, zeros, zeros]))
    ws.append(jnp.stack(vec_rows))

    out_lm = _decoder(q_lm, kv, ws)

    o0 = out_lm[:256].reshape(_BS, 64, _D)
    o1 = out_lm[256:1280].reshape(_BS, 256, _D)
    o2 = out_lm[1280:].reshape(_BS, 1024, _D)

    def back(o, h):
        return o.transpose(0, 2, 1).reshape(_BS, _D, h, h)

    return (back(o2, 32), back(o1, 16), back(o0, 8))


# speedup vs baseline: 31.5925x; 1.8099x over previous
import functools
import math

import jax
import jax.numpy as jnp
from jax import lax
from jax.experimental import pallas as pl
from jax.experimental.pallas import tpu as pltpu
from jax.experimental.pallas import tpu_sc as plsc

_D = 256
_NH = 8
_DH = 32
_K = 4
_DFF = 1024
_BS = 4
_NMEM = 5440
_NQB = 1344
_Q0 = _NMEM - _NQB
_NQ = _BS * _NQB
_BQ_SIM = 192
_SIM_BPB = _NQB // _BQ_SIM
_BD = 64
_NDB = _NQB // _BD
_NROWS = _NQ * _K
_NW = 32
_ROWS_PER_W = _NROWS // _NW
_CH = 168
_NCH = _ROWS_PER_W // _CH


def _mm(x, w):
    return lax.dot_general(x, w, (((1,), (1,)), ((), ())),
                           preferred_element_type=jnp.float32)


def _sim_topk_kernel(q_ref, mem_ref, idx_ref):
    q = q_ref[0]
    mem = mem_ref[0]
    s = _mm(q, mem)
    col = lax.broadcasted_iota(jnp.int32, s.shape, 1)
    picks = []
    for _ in range(_K):
        m = jnp.max(s, axis=1, keepdims=True)
        hit = s >= m
        idx_t = jnp.min(jnp.where(hit, col, jnp.int32(2**30)), axis=1)
        picks.append(idx_t[:, None])
        s = jnp.where(col == idx_t[:, None], jnp.float32(-3e38), s)
    idx_ref[0] = jnp.concatenate(picks, axis=1)


def _sim_topk(qs, all_feas):
    return pl.pallas_call(
        _sim_topk_kernel,
        grid=(_BS, _SIM_BPB),
        in_specs=[
            pl.BlockSpec((1, _BQ_SIM, _D), lambda b, k: (b, k, 0)),
            pl.BlockSpec((1, _NMEM, _D), lambda b, k: (b, 0, 0)),
        ],
        out_specs=pl.BlockSpec((1, _BQ_SIM, _K), lambda b, k: (b, k, 0)),
        out_shape=jax.ShapeDtypeStruct((_BS, _NQB, _K), jnp.int32),
    )(qs, all_feas)


def _sc_gather(table, idx_flat):
    mesh = plsc.VectorSubcoreMesh(core_axis_name="c", subcore_axis_name="s")

    @functools.partial(
        pl.kernel,
        mesh=mesh,
        out_type=jax.ShapeDtypeStruct((_NROWS, _D), jnp.float32),
        scratch_types=[
            pltpu.VMEM((_CH,), jnp.int32),
            pltpu.VMEM((_CH, _D), jnp.float32),
            pltpu.SemaphoreType.DMA,
        ],
    )
    def gather_k(table_hbm, idx_hbm, out_hbm, idx_v, rows_v, sem):
        wid = lax.axis_index("s") * 2 + lax.axis_index("c")
        base = wid * _ROWS_PER_W
        for c in range(_NCH):
            off = base + c * _CH
            pltpu.sync_copy(idx_hbm.at[pl.ds(off, _CH)], idx_v)
            pltpu.async_copy(table_hbm.at[idx_v], rows_v, sem).wait()
            pltpu.sync_copy(rows_v, out_hbm.at[pl.ds(off, _CH)])

    return gather_k(table, idx_flat)


def _ln(x, w, b):
    m = jnp.mean(x, axis=1, keepdims=True)
    d = x - m
    v = jnp.mean(d * d, axis=1, keepdims=True)
    return d / jnp.sqrt(v + 1e-5) * w + b


_NQD = _BS * _BD


def _dec_kernel(x_ref, kv_ref, sa_wv, sa_ow, ca_wq, ca_wk, ca_wv, ca_ow,
                l1w, l1b, l2w, vecs, o_ref):
    x = x_ref[...].reshape(_NQD, _D)
    V = vecs[0]
    bv_sa = V[0:1]; ob_sa = V[1:2]; n1w = V[2:3]; n1b = V[3:4]
    bq = V[4:5]; bk = V[5:6]; bv = V[6:7]; ob_ca = V[7:8]
    n2w = V[8:9]; n2b = V[9:10]; b_l2 = V[10:11]; n3w = V[11:12]; n3b = V[12:13]

    sa = _mm(_mm(x, sa_wv[0]) + bv_sa, sa_ow[0]) + ob_sa
    t1 = _ln(x + sa, n1w, n1b)

    q = _mm(t1, ca_wq[0]) + bq
    hrow = lax.broadcasted_iota(jnp.int32, (_D, _NH), 0) // _DH
    hcol = lax.broadcasted_iota(jnp.int32, (_D, _NH), 1)
    H = (hrow == hcol).astype(jnp.float32)
    scale = 1.0 / math.sqrt(_DH)
    ss, vs = [], []
    for j in range(_K):
        g = kv_ref[j].reshape(_NQD, _D)
        kp = _mm(g, ca_wk[0]) + bk
        vs.append(_mm(g, ca_wv[0]) + bv)
        sj = lax.dot_general(q * kp, H, (((1,), (0,)), ((), ())),
                             preferred_element_type=jnp.float32,
                             precision=lax.Precision.HIGHEST) * scale
        ss.append(sj)
    m = jnp.maximum(jnp.maximum(ss[0], ss[1]), jnp.maximum(ss[2], ss[3]))
    es = [jnp.exp(sj - m) for sj in ss]
    den = es[0] + es[1] + es[2] + es[3]
    att = jnp.zeros_like(x)
    for j in range(_K):
        wfull = lax.dot_general(es[j] / den, H, (((1,), (1,)), ((), ())),
                                preferred_element_type=jnp.float32,
                                precision=lax.Precision.HIGHEST)
        att = att + wfull * vs[j]
    ca = _mm(att, ca_ow[0]) + ob_ca
    t2 = _ln(t1 + ca, n2w, n2b)

    h = jnp.maximum(_mm(t2, l1w[0]) + l1b[0], 0.0)
    f = _mm(h, l2w[0]) + b_l2
    t3 = _ln(t2 + f, n3w, n3b)
    o_ref[...] = (t3 + x).reshape(_BS, _BD, _D)


def _lvl_of(j):
    return jnp.int32(2) - jnp.int32(j >= 16) - jnp.int32(j >= 20)


def _decoder(all_feas, kv, ws):
    def wspec(a, b):
        return pl.BlockSpec((1, a, b), lambda j: (_lvl_of(j), 0, 0))
    return pl.pallas_call(
        _dec_kernel,
        grid=(_NDB,),
        in_specs=[
            pl.BlockSpec((_BS, _BD, _D), lambda j: (0, _Q0 // _BD + j, 0)),
            pl.BlockSpec((_K, _BS, _BD, _D), lambda j: (0, 0, j, 0)),
            wspec(_D, _D), wspec(_D, _D), wspec(_D, _D), wspec(_D, _D),
            wspec(_D, _D), wspec(_D, _D),
            wspec(_DFF, _D), wspec(1, _DFF), wspec(_D, _DFF),
            wspec(16, _D),
        ],
        out_specs=pl.BlockSpec((_BS, _BD, _D), lambda j: (0, j, 0)),
        out_shape=jax.ShapeDtypeStruct((_BS, _NQB, _D), jnp.float32),
    )(all_feas, kv, *ws)


def kernel(src0, src1, src2, src3,
           L0_sa_in_w, L0_sa_in_b, L0_sa_out_w, L0_sa_out_b,
           L0_ca_in_w, L0_ca_in_b, L0_ca_out_w, L0_ca_out_b,
           L0_lin1_w, L0_lin1_b, L0_lin2_w, L0_lin2_b,
           L0_n1_w, L0_n1_b, L0_n2_w, L0_n2_b, L0_n3_w, L0_n3_b,
           L1_sa_in_w, L1_sa_in_b, L1_sa_out_w, L1_sa_out_b,
           L1_ca_in_w, L1_ca_in_b, L1_ca_out_w, L1_ca_out_b,
           L1_lin1_w, L1_lin1_b, L1_lin2_w, L1_lin2_b,
           L1_n1_w, L1_n1_b, L1_n2_w, L1_n2_b, L1_n3_w, L1_n3_b,
           L2_sa_in_w, L2_sa_in_b, L2_sa_out_w, L2_sa_out_b,
           L2_ca_in_w, L2_ca_in_b, L2_ca_out_w, L2_ca_out_b,
           L2_lin1_w, L2_lin1_b, L2_lin2_w, L2_lin2_b,
           L2_n1_w, L2_n1_b, L2_n2_w, L2_n2_b, L2_n3_w, L2_n3_b):
    srcs = [src0, src1, src2, src3]
    flat = [s.reshape(_BS, _D, -1).transpose(0, 2, 1) for s in srcs]
    all_feas = jnp.concatenate(flat, axis=1)
    qs = all_feas[:, _Q0:]

    idx = _sim_topk(qs, all_feas)
    idx_flat = idx.transpose(2, 0, 1).reshape(-1)

    kv = _sc_gather(all_feas[0], idx_flat).reshape(_K, _BS, _NQB, _D)

    def stack3(a, b, c):
        return jnp.stack([a, b, c])

    sa_in = (L0_sa_in_w, L1_sa_in_w, L2_sa_in_w)
    ca_in = (L0_ca_in_w, L1_ca_in_w, L2_ca_in_w)
    ws = [
        stack3(*[w[2 * _D:] for w in sa_in]),
        stack3(L0_sa_out_w, L1_sa_out_w, L2_sa_out_w),
        stack3(*[w[:_D] for w in ca_in]),
        stack3(*[w[_D:2 * _D] for w in ca_in]),
        stack3(*[w[2 * _D:] for w in ca_in]),
        stack3(L0_ca_out_w, L1_ca_out_w, L2_ca_out_w),
        stack3(L0_lin1_w, L1_lin1_w, L2_lin1_w),
        stack3(L0_lin1_b, L1_lin1_b, L2_lin1_b).reshape(3, 1, _DFF),
        stack3(L0_lin2_w, L1_lin2_w, L2_lin2_w),
    ]
    zeros = jnp.zeros((_D,), jnp.float32)
    vec_rows = []
    for sa_b, sa_ob, ca_b, ca_ob, l2b, n1w, n1b, n2w, n2b, n3w, n3b in (
        (L0_sa_in_b, L0_sa_out_b, L0_ca_in_b, L0_ca_out_b, L0_lin2_b,
         L0_n1_w, L0_n1_b, L0_n2_w, L0_n2_b, L0_n3_w, L0_n3_b),
        (L1_sa_in_b, L1_sa_out_b, L1_ca_in_b, L1_ca_out_b, L1_lin2_b,
         L1_n1_w, L1_n1_b, L1_n2_w, L1_n2_b, L1_n3_w, L1_n3_b),
        (L2_sa_in_b, L2_sa_out_b, L2_ca_in_b, L2_ca_out_b, L2_lin2_b,
         L2_n1_w, L2_n1_b, L2_n2_w, L2_n2_b, L2_n3_w, L2_n3_b),
    ):
        vec_rows.append(jnp.stack([
            sa_b[2 * _D:], sa_ob, n1w, n1b,
            ca_b[:_D], ca_b[_D:2 * _D], ca_b[2 * _D:], ca_ob,
            n2w, n2b, l2b, n3w, n3b, zeros, zeros, zeros]))
    ws.append(jnp.stack(vec_rows))

    out = _decoder(all_feas, kv, ws)

    def back(o, h):
        return o.transpose(0, 2, 1).reshape(_BS, _D, h, h)

    return (back(out[:, :1024], 32),
            back(out[:, 1024:1280], 16),
            back(out[:, 1280:], 8))


# speedup vs baseline: 33.2905x; 1.0537x over previous
import functools
import math

import jax
import jax.numpy as jnp
from jax import lax
from jax.experimental import pallas as pl
from jax.experimental.pallas import tpu as pltpu
from jax.experimental.pallas import tpu_sc as plsc

_D = 256
_NH = 8
_DH = 32
_K = 4
_DFF = 1024
_BS = 4
_HW = (4096, 1024, 256, 64)
_NMEM = 5440
_NQB = 1344
_NQP = 1408
_BQ_SIM = 352
_SIM_BPB = _NQP // _BQ_SIM
_BD = 128
_NDB = _NQP // _BD
_NROWS_B = _NQP * _K
_NW = 32
_CH = _NROWS_B // _NW


def _mm(x, w):
    return lax.dot_general(x, w, (((1,), (1,)), ((), ())),
                           preferred_element_type=jnp.float32)


def _sim_topk_kernel(q_ref, m0_ref, m1_ref, m2_ref, m3_ref, idx_ref):
    q = q_ref[0]
    parts = [
        lax.dot_general(q, m_ref[0], (((1,), (0,)), ((), ())),
                        preferred_element_type=jnp.float32)
        for m_ref in (m0_ref, m1_ref, m2_ref, m3_ref)
    ]
    s = jnp.concatenate(parts, axis=1)
    col = lax.broadcasted_iota(jnp.int32, s.shape, 1)
    picks = []
    for _ in range(_K):
        idx_t = jnp.argmax(s, axis=1).astype(jnp.int32)
        picks.append(idx_t[:, None])
        s = jnp.where(col == idx_t[:, None], jnp.float32(-3e38), s)
    idx_ref[0] = jnp.concatenate(picks, axis=1)


def _sim_topk_batch(qs, mems, b):
    return pl.pallas_call(
        _sim_topk_kernel,
        grid=(_SIM_BPB,),
        in_specs=[pl.BlockSpec((1, _BQ_SIM, _D), lambda k: (b, k, 0))] + [
            pl.BlockSpec((1, _D, hw), lambda k: (b, 0, 0)) for hw in _HW
        ],
        out_specs=pl.BlockSpec((1, _BQ_SIM, _K), lambda k: (0, k, 0)),
        out_shape=jax.ShapeDtypeStruct((1, _NQP, _K), jnp.int32),
    )(qs, *mems)


def _sc_gather_batch(table, idx_flat):
    mesh = plsc.VectorSubcoreMesh(core_axis_name="c", subcore_axis_name="s")

    @functools.partial(
        pl.kernel,
        mesh=mesh,
        out_type=jax.ShapeDtypeStruct((_NROWS_B, _D), jnp.float32),
        scratch_types=[
            pltpu.VMEM((_CH,), jnp.int32),
            pltpu.VMEM((_CH, _D), jnp.float32),
            pltpu.SemaphoreType.DMA,
        ],
    )
    def gather_k(table_hbm, idx_hbm, out_hbm, idx_v, rows_v, sem):
        wid = lax.axis_index("s") * 2 + lax.axis_index("c")
        base = wid * _CH
        pltpu.sync_copy(idx_hbm.at[pl.ds(base, _CH)], idx_v)
        pltpu.async_copy(table_hbm.at[idx_v], rows_v, sem).wait()
        pltpu.sync_copy(rows_v, out_hbm.at[pl.ds(base, _CH)])

    return gather_k(table, idx_flat)


def _ln(x, w, b):
    m = jnp.mean(x, axis=1, keepdims=True)
    d = x - m
    v = jnp.mean(d * d, axis=1, keepdims=True)
    return d / jnp.sqrt(v + 1e-5) * w + b


_NQD = _BS * _BD


def _dec_kernel(x_ref, kv0_ref, kv1_ref, kv2_ref, kv3_ref,
                sa_wv, sa_ow, ca_wq, ca_wk, ca_wv, ca_ow,
                l1w, l1b, l2w, vecs, o_ref):
    x = x_ref[...].reshape(_NQD, _D)
    kv_refs = (kv0_ref, kv1_ref, kv2_ref, kv3_ref)
    V = vecs[0]
    bv_sa = V[0:1]; ob_sa = V[1:2]; n1w = V[2:3]; n1b = V[3:4]
    bq = V[4:5]; bk = V[5:6]; bv = V[6:7]; ob_ca = V[7:8]
    n2w = V[8:9]; n2b = V[9:10]; b_l2 = V[10:11]; n3w = V[11:12]; n3b = V[12:13]

    sa = _mm(_mm(x, sa_wv[0]) + bv_sa, sa_ow[0]) + ob_sa
    t1 = _ln(x + sa, n1w, n1b)

    q = _mm(t1, ca_wq[0]) + bq
    hrow = lax.broadcasted_iota(jnp.int32, (_D, _NH), 0) // _DH
    hcol = lax.broadcasted_iota(jnp.int32, (_D, _NH), 1)
    H = (hrow == hcol).astype(jnp.float32)
    scale = 1.0 / math.sqrt(_DH)
    ss, vs = [], []
    for j in range(_K):
        g = jnp.concatenate([r[j] for r in kv_refs], axis=0)
        kp = _mm(g, ca_wk[0]) + bk
        vs.append(_mm(g, ca_wv[0]) + bv)
        sj = lax.dot_general(H, q * kp, (((0,), (1,)), ((), ())),
                             preferred_element_type=jnp.float32,
                             precision=lax.Precision.HIGHEST) * scale
        ss.append(sj)
    m = jnp.maximum(jnp.maximum(ss[0], ss[1]), jnp.maximum(ss[2], ss[3]))
    es = [jnp.exp(sj - m) for sj in ss]
    den = es[0] + es[1] + es[2] + es[3]
    att = jnp.zeros_like(x)
    for j in range(_K):
        wfull = lax.dot_general(es[j] / den, H, (((0,), (1,)), ((), ())),
                                preferred_element_type=jnp.float32,
                                precision=lax.Precision.HIGHEST)
        att = att + wfull * vs[j]
    ca = _mm(att, ca_ow[0]) + ob_ca
    t2 = _ln(t1 + ca, n2w, n2b)

    h = jnp.maximum(_mm(t2, l1w[0]) + l1b[0], 0.0)
    f = _mm(h, l2w[0]) + b_l2
    t3 = _ln(t2 + f, n3w, n3b)
    y = (t3 + x).reshape(_BS, _BD, _D)
    j = pl.program_id(0)
    o_ref[:, :, pl.ds(j, 1), :] = jnp.swapaxes(y, 1, 2).reshape(_BS, _D, 1, _BD)


def _lvl_of(j):
    return jnp.int32(2) - jnp.int32(j >= 8) - jnp.int32(j >= 10)


def _decoder(qs, kvs, ws):
    def wspec(a, b):
        return pl.BlockSpec((1, a, b), lambda j: (_lvl_of(j), 0, 0))
    kvspec = pl.BlockSpec((_K, _BD, _D), lambda j: (0, j, 0))
    return pl.pallas_call(
        _dec_kernel,
        grid=(_NDB,),
        in_specs=[
            pl.BlockSpec((_BS, _BD, _D), lambda j: (0, j, 0)),
            kvspec, kvspec, kvspec, kvspec,
            wspec(_D, _D), wspec(_D, _D), wspec(_D, _D), wspec(_D, _D),
            wspec(_D, _D), wspec(_D, _D),
            wspec(_DFF, _D), wspec(1, _DFF), wspec(_D, _DFF),
            wspec(16, _D),
        ],
        out_specs=pl.BlockSpec((_BS, _D, _NDB, _BD), lambda j: (0, 0, 0, 0)),
        out_shape=jax.ShapeDtypeStruct((_BS, _D, _NDB, _BD), jnp.float32),
    )(qs, *kvs, *ws)


def kernel(src0, src1, src2, src3,
           L0_sa_in_w, L0_sa_in_b, L0_sa_out_w, L0_sa_out_b,
           L0_ca_in_w, L0_ca_in_b, L0_ca_out_w, L0_ca_out_b,
           L0_lin1_w, L0_lin1_b, L0_lin2_w, L0_lin2_b,
           L0_n1_w, L0_n1_b, L0_n2_w, L0_n2_b, L0_n3_w, L0_n3_b,
           L1_sa_in_w, L1_sa_in_b, L1_sa_out_w, L1_sa_out_b,
           L1_ca_in_w, L1_ca_in_b, L1_ca_out_w, L1_ca_out_b,
           L1_lin1_w, L1_lin1_b, L1_lin2_w, L1_lin2_b,
           L1_n1_w, L1_n1_b, L1_n2_w, L1_n2_b, L1_n3_w, L1_n3_b,
           L2_sa_in_w, L2_sa_in_b, L2_sa_out_w, L2_sa_out_b,
           L2_ca_in_w, L2_ca_in_b, L2_ca_out_w, L2_ca_out_b,
           L2_lin1_w, L2_lin1_b, L2_lin2_w, L2_lin2_b,
           L2_n1_w, L2_n1_b, L2_n2_w, L2_n2_b, L2_n3_w, L2_n3_b):
    mems = [s.reshape(_BS, _D, -1) for s in (src0, src1, src2, src3)]
    flatq = [m.transpose(0, 2, 1) for m in mems[1:]]
    pad = jnp.zeros((_BS, _NQP - _NQB, _D), jnp.float32)
    qs = jnp.concatenate(flatq + [pad], axis=1)
    table0 = jnp.concatenate(
        [mems[0][0].T] + [f[0] for f in flatq], axis=0)

    kvs = []
    for b in range(_BS):
        idx_b = _sim_topk_batch(qs, mems, b)
        flat_b = idx_b[0].T.reshape(-1)
        kvs.append(_sc_gather_batch(table0, flat_b).reshape(_K, _NQP, _D))

    def stack3(a, b, c):
        return jnp.stack([a, b, c])

    sa_in = (L0_sa_in_w, L1_sa_in_w, L2_sa_in_w)
    ca_in = (L0_ca_in_w, L1_ca_in_w, L2_ca_in_w)
    ws = [
        stack3(*[w[2 * _D:] for w in sa_in]),
        stack3(L0_sa_out_w, L1_sa_out_w, L2_sa_out_w),
        stack3(*[w[:_D] for w in ca_in]),
        stack3(*[w[_D:2 * _D] for w in ca_in]),
        stack3(*[w[2 * _D:] for w in ca_in]),
        stack3(L0_ca_out_w, L1_ca_out_w, L2_ca_out_w),
        stack3(L0_lin1_w, L1_lin1_w, L2_lin1_w),
        stack3(L0_lin1_b, L1_lin1_b, L2_lin1_b).reshape(3, 1, _DFF),
        stack3(L0_lin2_w, L1_lin2_w, L2_lin2_w),
    ]
    zeros = jnp.zeros((_D,), jnp.float32)
    vec_rows = []
    for sa_b, sa_ob, ca_b, ca_ob, l2b, n1w, n1b, n2w, n2b, n3w, n3b in (
        (L0_sa_in_b, L0_sa_out_b, L0_ca_in_b, L0_ca_out_b, L0_lin2_b,
         L0_n1_w, L0_n1_b, L0_n2_w, L0_n2_b, L0_n3_w, L0_n3_b),
        (L1_sa_in_b, L1_sa_out_b, L1_ca_in_b, L1_ca_out_b, L1_lin2_b,
         L1_n1_w, L1_n1_b, L1_n2_w, L1_n2_b, L1_n3_w, L1_n3_b),
        (L2_sa_in_b, L2_sa_out_b, L2_ca_in_b, L2_ca_out_b, L2_lin2_b,
         L2_n1_w, L2_n1_b, L2_n2_w, L2_n2_b, L2_n3_w, L2_n3_b),
    ):
        vec_rows.append(jnp.stack([
            sa_b[2 * _D:], sa_ob, n1w, n1b,
            ca_b[:_D], ca_b[_D:2 * _D], ca_b[2 * _D:], ca_ob,
            n2w, n2b, l2b, n3w, n3b, zeros, zeros, zeros]))
    ws.append(jnp.stack(vec_rows))

    out = _decoder(qs, kvs, ws).reshape(_BS, _D, _NQP)

    return (out[:, :, :1024].reshape(_BS, _D, 32, 32),
            out[:, :, 1024:1280].reshape(_BS, _D, 16, 16),
            out[:, :, 1280:1344].reshape(_BS, _D, 8, 8))
